# baseline (speedup 1.0000x reference)
import jax
import jax.numpy as jnp
from jax import lax
from jax.experimental import pallas as pl
from jax.experimental.pallas import tpu as pltpu
from jax.experimental.pallas import tpu_sc as plsc

NC = 2
NS = 16
L = 16
NW = NC * NS

BATCH = 16384
D = 32
B_PER_W = BATCH // NW
CHUNK = 128
N_CHUNKS = B_PER_W // CHUNK
N_GROUPS = B_PER_W // L


def _gmf_body(u_tab, i_tab, u_idx, i_idx, wb,
              out,
              u_idx_v, i_idx_v, u_rows, i_rows, wb_v, prod_t, out_v, sem):
    wid = lax.axis_index("s") * NC + lax.axis_index("c")
    cbase = wid * N_CHUNKS

    pltpu.sync_copy(u_idx.at[pl.ds(cbase, N_CHUNKS)], u_idx_v)
    pltpu.sync_copy(i_idx.at[pl.ds(cbase, N_CHUNKS)], i_idx_v)
    pltpu.sync_copy(wb, wb_v)

    copies = []
    for k in range(N_CHUNKS):
        copies.append(pltpu.async_copy(
            u_tab.at[u_idx_v.at[k]], u_rows.at[pl.ds(k * CHUNK, CHUNK)], sem))
        copies.append(pltpu.async_copy(
            i_tab.at[i_idx_v.at[k]], i_rows.at[pl.ds(k * CHUNK, CHUNK)], sem))
    for c in copies:
        c.wait()

    w_lo = wb_v[0]
    w_hi = wb_v[1]
    b_vec = wb_v[2]
    lane = jnp.arange(L, dtype=jnp.int32)
    sc_lo = lane * B_PER_W
    sc_hi = sc_lo + L * B_PER_W

    def row(r, carry):
        p_lo = u_rows[r, pl.ds(0, L)] * i_rows[r, pl.ds(0, L)] * w_lo
        p_hi = u_rows[r, pl.ds(L, L)] * i_rows[r, pl.ds(L, L)] * w_hi
        plsc.store_scatter(prod_t, [sc_lo + r], p_lo)
        plsc.store_scatter(prod_t, [sc_hi + r], p_hi)
        return carry

    lax.fori_loop(0, B_PER_W, row, 0)

    def group(g, carry):
        base = g * L
        acc = b_vec
        for d in range(D):
            acc = acc + prod_t[pl.ds(d * B_PER_W + base, L)]
        out_v[pl.ds(base, L)] = 1.0 / (1.0 + jnp.exp(-acc))
        return carry

    lax.fori_loop(0, N_GROUPS, group, 0)

    pltpu.sync_copy(out_v, out.at[pl.ds(wid * B_PER_W, B_PER_W)])


_gmf = pl.kernel(
    _gmf_body,
    out_type=jax.ShapeDtypeStruct((BATCH,), jnp.float32),
    mesh=plsc.VectorSubcoreMesh(core_axis_name="c", subcore_axis_name="s"),
    compiler_params=pltpu.CompilerParams(
        needs_layout_passes=False, use_tc_tiling_on_sc=False),
    scratch_types=[
        pltpu.VMEM((N_CHUNKS, CHUNK), jnp.int32),
        pltpu.VMEM((N_CHUNKS, CHUNK), jnp.int32),
        pltpu.VMEM((B_PER_W, D), jnp.float32),
        pltpu.VMEM((B_PER_W, D), jnp.float32),
        pltpu.VMEM((3, L), jnp.float32),
        pltpu.VMEM((B_PER_W * D,), jnp.float32),
        pltpu.VMEM((B_PER_W,), jnp.float32),
        pltpu.SemaphoreType.DMA,
    ],
)


@jax.jit
def kernel(user_input, item_input, user_table, item_table, linear_w, linear_b):
    u_idx = user_input.astype(jnp.int32).reshape(NW * N_CHUNKS, CHUNK)
    i_idx = item_input.astype(jnp.int32).reshape(NW * N_CHUNKS, CHUNK)
    w = linear_w.reshape(D)
    wb = jnp.stack([w[:L], w[L:], jnp.broadcast_to(linear_b, (L,))])
    out = _gmf(user_table, item_table, u_idx, i_idx, wb)
    return out.reshape(BATCH, 1)
